# baseline (speedup 1.0000x reference)
import functools

import jax
import jax.numpy as jnp
from jax import lax
from jax.experimental import pallas as pl
from jax.experimental.pallas import tpu as pltpu
from jax.experimental.pallas import tpu_sc as plsc

D = 128
NC, NS = 2, 16
NW = NC * NS
CHUNK = 128
NBUF = 2
LANES = 16


def _sc_aggregate(x, src_p, dst_p, n_nodes, e_per_tile):
    n_chunks = e_per_tile // CHUNK
    assert n_chunks % NBUF == 0
    n_acc = ((n_nodes + 1 + NS * CHUNK - 1) // (NS * CHUNK)) * (NS * CHUNK)
    rows_per_tile = n_acc // NS
    n_pieces = rows_per_tile // CHUNK
    mesh = plsc.VectorSubcoreMesh(core_axis_name="c", subcore_axis_name="s")

    @functools.partial(
        pl.kernel,
        out_type=(
            jax.ShapeDtypeStruct((NC, n_acc, D), jnp.float32),
            jax.ShapeDtypeStruct((NW, n_acc), jnp.float32),
        ),
        mesh=mesh,
        scratch_types=(
            [pltpu.VMEM_SHARED((n_acc, D), jnp.float32)]
            + [pltpu.VMEM((n_acc,), jnp.float32)]
            + [pltpu.VMEM((CHUNK,), jnp.int32) for _ in range(2 * NBUF)]
            + [pltpu.VMEM((CHUNK, D), jnp.float32) for _ in range(NBUF)]
            + [pltpu.SemaphoreType.DMA for _ in range(NBUF)]
        ),
        compiler_params=pltpu.CompilerParams(needs_layout_passes=False),
    )
    def agg(x_hbm, src_hbm, dst_hbm, out_hbm, deg_hbm, acc, hist, *bufs):
        srcs = bufs[:NBUF]
        dsts = bufs[NBUF:2 * NBUF]
        rows = bufs[2 * NBUF:3 * NBUF]
        sems = bufs[3 * NBUF:4 * NBUF]
        c = lax.axis_index("c")
        s = lax.axis_index("s")

        zv = jnp.zeros((LANES,), jnp.float32)

        def fill_hist(i, carry):
            hist[pl.ds(i * LANES, LANES)] = zv
            return carry

        lax.fori_loop(0, n_acc // LANES, fill_hist, 0)

        def fill(i, carry):
            for j in range(D // LANES):
                rows[0][i, pl.ds(j * LANES, LANES)] = zv
            return carry

        lax.fori_loop(0, CHUNK, fill, 0)
        r0 = s * rows_per_tile
        for k in range(n_pieces):
            pltpu.sync_copy(rows[0], acc.at[pl.ds(r0 + k * CHUNK, CHUNK)])
        plsc.subcore_barrier()

        wid = c * NS + s
        ebase = wid * e_per_tile
        ones16 = jnp.ones((LANES,), jnp.float32)

        for b in range(NBUF):
            eb = ebase + b * CHUNK
            pltpu.sync_copy(src_hbm.at[pl.ds(eb, CHUNK)], srcs[b])
            pltpu.sync_copy(dst_hbm.at[pl.ds(eb, CHUNK)], dsts[b])
            pltpu.async_copy(x_hbm.at[srcs[b]], rows[b], sems[b])

        def outer(t, carry):
            for b in range(NBUF):
                g = t * NBUF + b
                pltpu.make_async_copy(x_hbm.at[srcs[b]], rows[b], sems[b]).wait()
                pltpu.sync_copy(rows[b], acc.at[dsts[b]], add=True)
                for j in range(CHUNK // LANES):
                    idx16 = dsts[b][pl.ds(j * LANES, LANES)]
                    plsc.addupdate_scatter(hist, [idx16], ones16)

                @pl.when(g + NBUF < n_chunks)
                def _():
                    eb = ebase + (g + NBUF) * CHUNK
                    pltpu.sync_copy(src_hbm.at[pl.ds(eb, CHUNK)], srcs[b])
                    pltpu.sync_copy(dst_hbm.at[pl.ds(eb, CHUNK)], dsts[b])
                    pltpu.async_copy(x_hbm.at[srcs[b]], rows[b], sems[b])

            return carry

        lax.fori_loop(0, n_chunks // NBUF, outer, 0)
        plsc.subcore_barrier()

        for k in range(n_pieces):
            pltpu.sync_copy(acc.at[pl.ds(r0 + k * CHUNK, CHUNK)], rows[0])
            pltpu.sync_copy(rows[0], out_hbm.at[c, pl.ds(r0 + k * CHUNK, CHUNK)])
        pltpu.sync_copy(hist, deg_hbm.at[wid])

    return agg(x, src_p, dst_p)


def _tc_head(x, partial, degs, W, b):
    n = x.shape[0]
    R = 1024
    grid = ((n + R - 1) // R,)

    def body(x_ref, p_ref, d_ref, w_ref, b_ref, o_ref):
        xb = x_ref[...]
        p = p_ref[...]
        ssum = p[0] + p[1]
        deg = jnp.sum(d_ref[...], axis=0)[:, None]
        mean = ssum / jnp.maximum(deg, 1.0)
        w = w_ref[...]
        h = (
            jnp.dot(xb, w[:D], preferred_element_type=jnp.float32,
                    precision=lax.Precision.HIGHEST)
            + jnp.dot(mean, w[D:], preferred_element_type=jnp.float32,
                      precision=lax.Precision.HIGHEST)
            + b_ref[...]
        )
        h = jnp.maximum(h, 0.0)
        nrm = jnp.sqrt(jnp.sum(h * h, axis=1, keepdims=True))
        o_ref[...] = h / jnp.maximum(nrm, 1e-12)

    return pl.pallas_call(
        body,
        grid=grid,
        in_specs=[
            pl.BlockSpec((R, D), lambda i: (i, 0)),
            pl.BlockSpec((NC, R, D), lambda i: (0, i, 0)),
            pl.BlockSpec((NW, R), lambda i: (0, i)),
            pl.BlockSpec((2 * D, D), lambda i: (0, 0)),
            pl.BlockSpec((1, D), lambda i: (0, 0)),
        ],
        out_specs=pl.BlockSpec((R, D), lambda i: (i, 0)),
        out_shape=jax.ShapeDtypeStruct((n, D), jnp.float32),
    )(x, partial, degs, W, b.reshape(1, D))


def kernel(input_matrix, adjacency_coo_matrix, W, b):
    x = input_matrix
    n = x.shape[0]
    e = adjacency_coo_matrix.shape[1]
    per_tile_chunks = (e + NW * CHUNK - 1) // (NW * CHUNK)
    per_tile_chunks = ((per_tile_chunks + NBUF - 1) // NBUF) * NBUF
    e_per_tile = per_tile_chunks * CHUNK
    e_pad = NW * e_per_tile
    pad = e_pad - e
    src = adjacency_coo_matrix[0].astype(jnp.int32)
    dst = adjacency_coo_matrix[1].astype(jnp.int32)
    n_acc = ((n + 1 + NS * CHUNK - 1) // (NS * CHUNK)) * (NS * CHUNK)
    pad_i = jnp.arange(pad, dtype=jnp.int32)
    src_p = jnp.concatenate([src, pad_i % jnp.int32(n)])
    dst_p = jnp.concatenate([dst, n + pad_i % jnp.int32(n_acc - n)])
    partial, degs = _sc_aggregate(x, src_p, dst_p, n, e_per_tile)
    return _tc_head(x, partial, degs, W, b)


# speedup vs baseline: 1.2424x; 1.2424x over previous
import functools

import jax
import jax.numpy as jnp
from jax import lax
from jax.experimental import pallas as pl
from jax.experimental.pallas import tpu as pltpu
from jax.experimental.pallas import tpu_sc as plsc

D = 128
NC, NS = 2, 16
NW = NC * NS
CHUNK = 128
NBUF = 2
LANES = 16


def _sc_aggregate(x, adj, n_nodes):
    e = adj.shape[1]
    assert e % CHUNK == 0
    n_chunks = e // CHUNK
    base_chunks = n_chunks // NW
    extra = n_chunks - base_chunks * NW
    n_acc = ((n_nodes + NS * CHUNK - 1) // (NS * CHUNK)) * (NS * CHUNK)
    rows_per_tile = n_acc // NS
    n_pieces = rows_per_tile // CHUNK
    mesh = plsc.VectorSubcoreMesh(core_axis_name="c", subcore_axis_name="s")

    @functools.partial(
        pl.kernel,
        out_type=(
            jax.ShapeDtypeStruct((NC, n_acc, D), jnp.float32),
            jax.ShapeDtypeStruct((NW, n_acc), jnp.float32),
        ),
        mesh=mesh,
        scratch_types=(
            [pltpu.VMEM_SHARED((n_acc, D), jnp.float32)]
            + [pltpu.VMEM((n_acc,), jnp.float32)]
            + [pltpu.VMEM((2, CHUNK), jnp.int32) for _ in range(NBUF)]
            + [pltpu.VMEM((CHUNK, D), jnp.float32) for _ in range(NBUF)]
            + [pltpu.SemaphoreType.DMA for _ in range(NBUF)]
        ),
        compiler_params=pltpu.CompilerParams(needs_layout_passes=False),
    )
    def agg(x_hbm, adj_hbm, out_hbm, deg_hbm, acc, hist, *bufs):
        eidxs = bufs[:NBUF]
        rows = bufs[NBUF:2 * NBUF]
        sems = bufs[2 * NBUF:3 * NBUF]
        c = lax.axis_index("c")
        s = lax.axis_index("s")

        zvf = jnp.zeros((LANES,), jnp.float32)

        def fill_hist(i, carry):
            hist[pl.ds(i * LANES, LANES)] = zvf
            return carry

        lax.fori_loop(0, n_acc // LANES, fill_hist, 0)

        def fill(i, carry):
            for j in range(D // LANES):
                rows[0][i, pl.ds(j * LANES, LANES)] = zvf
            return carry

        lax.fori_loop(0, CHUNK, fill, 0)
        r0 = s * rows_per_tile
        for k in range(n_pieces):
            pltpu.sync_copy(rows[0], acc.at[pl.ds(r0 + k * CHUNK, CHUNK)])
        plsc.subcore_barrier()

        wid = c * NS + s
        chunk0 = wid * base_chunks + jnp.minimum(wid, extra)
        ones16 = jnp.ones((LANES,), jnp.float32)

        def process(eidx, row, sem):
            pltpu.make_async_copy(x_hbm.at[eidx.at[0]], row, sem).wait()
            pltpu.sync_copy(row, acc.at[eidx.at[1]], add=True)
            for j in range(CHUNK // LANES):
                idx16 = eidx[1, pl.ds(j * LANES, LANES)]
                plsc.addupdate_scatter(hist, [idx16], ones16)

        assert base_chunks % NBUF == 0
        for b in range(NBUF):
            pltpu.sync_copy(adj_hbm.at[:, pl.ds((chunk0 + b) * CHUNK, CHUNK)],
                            eidxs[b])
            pltpu.async_copy(x_hbm.at[eidxs[b].at[0]], rows[b], sems[b])

        def outer(t, carry):
            for b in range(NBUF):
                g = t * NBUF + b
                eidx, row, sem = eidxs[b], rows[b], sems[b]
                process(eidx, row, sem)

                @pl.when(g + NBUF < base_chunks)
                def _():
                    cb = (chunk0 + g + NBUF) * CHUNK
                    pltpu.sync_copy(adj_hbm.at[:, pl.ds(cb, CHUNK)], eidx)
                    pltpu.async_copy(x_hbm.at[eidx.at[0]], row, sem)

            return carry

        lax.fori_loop(0, base_chunks // NBUF, outer, 0)

        @pl.when(wid < extra)
        def _():
            cb = (chunk0 + base_chunks) * CHUNK
            pltpu.sync_copy(adj_hbm.at[:, pl.ds(cb, CHUNK)], eidxs[0])
            pltpu.async_copy(x_hbm.at[eidxs[0].at[0]], rows[0], sems[0])
            process(eidxs[0], rows[0], sems[0])

        plsc.subcore_barrier()

        for k in range(n_pieces):
            pltpu.sync_copy(acc.at[pl.ds(r0 + k * CHUNK, CHUNK)], rows[0])
            pltpu.sync_copy(rows[0], out_hbm.at[c, pl.ds(r0 + k * CHUNK, CHUNK)])
        pltpu.sync_copy(hist, deg_hbm.at[wid])

    return agg(x, adj)


def _tc_head(x, partial, degs, W, b):
    n = x.shape[0]
    R = 1024
    grid = ((n + R - 1) // R,)

    def body(x_ref, p_ref, d_ref, w_ref, b_ref, o_ref):
        xb = x_ref[...]
        p = p_ref[...]
        ssum = p[0] + p[1]
        deg = jnp.sum(d_ref[...], axis=0)[:, None]
        mean = ssum / jnp.maximum(deg, 1.0)
        w = w_ref[...]
        h = (
            jnp.dot(xb, w[:D], preferred_element_type=jnp.float32,
                    precision=lax.Precision.HIGHEST)
            + jnp.dot(mean, w[D:], preferred_element_type=jnp.float32,
                      precision=lax.Precision.HIGHEST)
            + b_ref[...]
        )
        h = jnp.maximum(h, 0.0)
        nrm = jnp.sqrt(jnp.sum(h * h, axis=1, keepdims=True))
        o_ref[...] = h / jnp.maximum(nrm, 1e-12)

    return pl.pallas_call(
        body,
        grid=grid,
        in_specs=[
            pl.BlockSpec((R, D), lambda i: (i, 0)),
            pl.BlockSpec((NC, R, D), lambda i: (0, i, 0)),
            pl.BlockSpec((NW, R), lambda i: (0, i)),
            pl.BlockSpec((2 * D, D), lambda i: (0, 0)),
            pl.BlockSpec((1, D), lambda i: (0, 0)),
        ],
        out_specs=pl.BlockSpec((R, D), lambda i: (i, 0)),
        out_shape=jax.ShapeDtypeStruct((n, D), jnp.float32),
    )(x, partial, degs, W, b.reshape(1, D))


def kernel(input_matrix, adjacency_coo_matrix, W, b):
    x = input_matrix
    n = x.shape[0]
    adj = adjacency_coo_matrix.astype(jnp.int32)
    partial, degs = _sc_aggregate(x, adj, n)
    return _tc_head(x, partial, degs, W, b)


# speedup vs baseline: 1.4324x; 1.1529x over previous
import functools

import jax
import jax.numpy as jnp
from jax import lax
from jax.experimental import pallas as pl
from jax.experimental.pallas import tpu as pltpu
from jax.experimental.pallas import tpu_sc as plsc

D = 128
NC, NS = 2, 16
NW = NC * NS
CHUNK = 128
NBUF = 2
LANES = 16


def _sc_aggregate(x, adj, n_nodes):
    e = adj.shape[1]
    assert e % CHUNK == 0
    n_chunks = e // CHUNK
    U = 4
    base_rounds = n_chunks // NW // U
    rem = n_chunks - base_rounds * U * NW
    assert rem % U == 0, "edge count must split into U-chunk rounds"
    extra_tiles = rem // U
    assert extra_tiles <= NW and base_rounds >= 1
    n_acc = ((n_nodes + NS * CHUNK - 1) // (NS * CHUNK)) * (NS * CHUNK)
    rows_per_tile = n_acc // NS
    n_pieces = rows_per_tile // CHUNK
    mesh = plsc.VectorSubcoreMesh(core_axis_name="c", subcore_axis_name="s")

    @functools.partial(
        pl.kernel,
        out_type=(
            jax.ShapeDtypeStruct((NC, n_acc, D), jnp.float32),
            jax.ShapeDtypeStruct((NW, n_acc), jnp.float32),
        ),
        mesh=mesh,
        scratch_types=(
            [pltpu.VMEM_SHARED((n_acc, D), jnp.float32)]
            + [pltpu.VMEM((n_acc,), jnp.float32)]
            + [pltpu.VMEM((2, CHUNK), jnp.int32) for _ in range(2 * NBUF)]
            + [pltpu.VMEM((CHUNK, D), jnp.float32) for _ in range(NBUF)]
            + [pltpu.SemaphoreType.DMA for _ in range(2 * NBUF)]
        ),
        compiler_params=pltpu.CompilerParams(needs_layout_passes=False),
    )
    def agg(x_hbm, adj_hbm, out_hbm, deg_hbm, acc, hist, *bufs):
        eidxs = bufs[:2 * NBUF]
        rows = bufs[2 * NBUF:3 * NBUF]
        gsems = bufs[3 * NBUF:4 * NBUF]
        ssems = bufs[4 * NBUF:5 * NBUF]
        c = lax.axis_index("c")
        s = lax.axis_index("s")

        zvf = jnp.zeros((LANES,), jnp.float32)

        def fill_hist(i, carry):
            hist[pl.ds(i * LANES, LANES)] = zvf
            return carry

        lax.fori_loop(0, n_acc // LANES, fill_hist, 0)

        def fill(i, carry):
            for j in range(D // LANES):
                rows[0][i, pl.ds(j * LANES, LANES)] = zvf
            return carry

        lax.fori_loop(0, CHUNK, fill, 0)
        r0 = s * rows_per_tile
        for k in range(n_pieces):
            pltpu.sync_copy(rows[0], acc.at[pl.ds(r0 + k * CHUNK, CHUNK)])
        plsc.subcore_barrier()

        wid = c * NS + s
        chunk0 = wid * base_rounds * U + U * jnp.minimum(wid, extra_tiles)
        my_chunks = U * (base_rounds + jnp.where(wid < extra_tiles, 1, 0))
        ones16 = jnp.ones((LANES,), jnp.float32)

        for q in range(NBUF):
            pltpu.sync_copy(adj_hbm.at[:, pl.ds((chunk0 + q) * CHUNK, CHUNK)],
                            eidxs[q])
            pltpu.async_copy(x_hbm.at[eidxs[q].at[0]], rows[q], gsems[q])

        def outer(t, carry):
            for b in range(U):
                g = t * U + b
                r = b % NBUF
                eidx, row = eidxs[b % (2 * NBUF)], rows[r]
                pltpu.make_async_copy(x_hbm.at[eidx.at[0]], row, gsems[r]).wait()
                scat = pltpu.async_copy(row, acc.at[eidx.at[1]], ssems[r],
                                        add=True)
                for j in range(CHUNK // LANES):
                    idx16 = eidx[1, pl.ds(j * LANES, LANES)]
                    plsc.addupdate_scatter(hist, [idx16], ones16)

                @pl.when(g + NBUF < my_chunks)
                def _():
                    nq = (b + NBUF) % (2 * NBUF)
                    cb = (chunk0 + g + NBUF) * CHUNK
                    pltpu.sync_copy(adj_hbm.at[:, pl.ds(cb, CHUNK)], eidxs[nq])
                    scat.wait()
                    pltpu.async_copy(x_hbm.at[eidxs[nq].at[0]], row, gsems[r])

                @pl.when(g + NBUF >= my_chunks)
                def _():
                    scat.wait()

            return carry

        lax.fori_loop(0, base_rounds + jnp.where(wid < extra_tiles, 1, 0),
                      outer, 0)
        plsc.subcore_barrier()

        for k in range(n_pieces):
            pltpu.sync_copy(acc.at[pl.ds(r0 + k * CHUNK, CHUNK)], rows[0])
            pltpu.sync_copy(rows[0], out_hbm.at[c, pl.ds(r0 + k * CHUNK, CHUNK)])
        pltpu.sync_copy(hist, deg_hbm.at[wid])

    return agg(x, adj)


def _tc_head(x, partial, degs, W, b):
    n = x.shape[0]
    R = 1024
    grid = ((n + R - 1) // R,)

    def body(x_ref, p_ref, d_ref, w_ref, b_ref, o_ref):
        xb = x_ref[...]
        p = p_ref[...]
        ssum = p[0] + p[1]
        deg = jnp.sum(d_ref[...], axis=0)[:, None]
        mean = ssum / jnp.maximum(deg, 1.0)
        w = w_ref[...]
        h = (
            jnp.dot(xb, w[:D], preferred_element_type=jnp.float32)
            + jnp.dot(mean, w[D:], preferred_element_type=jnp.float32)
            + b_ref[...]
        )
        h = jnp.maximum(h, 0.0)
        nrm = jnp.sqrt(jnp.sum(h * h, axis=1, keepdims=True))
        o_ref[...] = h / jnp.maximum(nrm, 1e-12)

    return pl.pallas_call(
        body,
        grid=grid,
        in_specs=[
            pl.BlockSpec((R, D), lambda i: (i, 0)),
            pl.BlockSpec((NC, R, D), lambda i: (0, i, 0)),
            pl.BlockSpec((NW, R), lambda i: (0, i)),
            pl.BlockSpec((2 * D, D), lambda i: (0, 0)),
            pl.BlockSpec((1, D), lambda i: (0, 0)),
        ],
        out_specs=pl.BlockSpec((R, D), lambda i: (i, 0)),
        out_shape=jax.ShapeDtypeStruct((n, D), jnp.float32),
    )(x, partial, degs, W, b.reshape(1, D))


def kernel(input_matrix, adjacency_coo_matrix, W, b):
    x = input_matrix
    n = x.shape[0]
    adj = adjacency_coo_matrix.astype(jnp.int32)
    partial, degs = _sc_aggregate(x, adj, n)
    return _tc_head(x, partial, degs, W, b)
